# baseline (speedup 1.0000x reference)
import functools
import jax
import jax.numpy as jnp
from jax import lax
from jax.experimental import pallas as pl
from jax.experimental.pallas import tpu as pltpu

N = 10000
E = 320000
C = 128
H = 256
L = 128
EB = 1280
NB = 1000


def _edge_mlp_kernel(row_ref, col_ref, x_ref, pos_ref,
                     w1a_ref, w1b_ref, b1_ref, w2_ref, b2_ref, w3_ref, b3_ref,
                     out_ref, feat_ref, pc_ref):
    def body(i, _):
        r = row_ref[0, i]
        c = col_ref[0, i]
        feat_ref[pl.ds(i, 1), :] = x_ref[pl.ds(r, 1), :]
        pc_ref[pl.ds(i, 1), :] = pos_ref[pl.ds(c, 1), :]
        return 0

    lax.fori_loop(0, EB, body, 0)
    feat = feat_ref[...]
    ea = feat[:, :16] - pc_ref[...]
    ea = jnp.where(ea > 0.5, ea - 1.0, ea)
    ea = jnp.where(-ea > 0.5, ea + 1.0, ea)
    h = jnp.dot(feat, w1a_ref[...], preferred_element_type=jnp.float32)
    h = h + jnp.dot(ea, w1b_ref[...], preferred_element_type=jnp.float32)
    h = jax.nn.relu(h + b1_ref[...])
    h = jax.nn.relu(jnp.dot(h, w2_ref[...], preferred_element_type=jnp.float32)
                    + b2_ref[...])
    out_ref[...] = (jnp.dot(h, w3_ref[...], preferred_element_type=jnp.float32)
                    + b3_ref[...])


def _segment_kernel(col_ref, val_ref, s_ref, m_ref, c_ref):
    @pl.when(pl.program_id(0) == 0)
    def _():
        s_ref[...] = jnp.zeros_like(s_ref)
        m_ref[...] = jnp.full_like(m_ref, -3.0e38)
        c_ref[...] = jnp.zeros_like(c_ref)

    def body(i, _):
        c = col_ref[0, i]
        v = val_ref[pl.ds(i, 1), :]
        s_ref[pl.ds(c, 1), :] += v
        m_ref[pl.ds(c, 1), :] = jnp.maximum(m_ref[pl.ds(c, 1), :], v)
        c_ref[pl.ds(c, 1), :] += 1.0
        return 0

    lax.fori_loop(0, EB, body, 0)


def _node_mlp_kernel(x_ref, s_ref, m_ref, c_ref,
                     wa_ref, wb_ref, wc_ref, wd_ref, b1_ref,
                     w2_ref, b2_ref, w3_ref, b3_ref, out_ref):
    cnt = c_ref[...]
    s = s_ref[...]
    mean = s / jnp.maximum(cnt, 1.0)
    mx = jnp.where(cnt > 0, m_ref[...], 0.0)
    h = jnp.dot(x_ref[...], wa_ref[...], preferred_element_type=jnp.float32)
    h = h + jnp.dot(mean, wb_ref[...], preferred_element_type=jnp.float32)
    h = h + jnp.dot(mx, wc_ref[...], preferred_element_type=jnp.float32)
    h = h + jnp.dot(s, wd_ref[...], preferred_element_type=jnp.float32)
    h = jax.nn.relu(h + b1_ref[...])
    h = jax.nn.relu(jnp.dot(h, w2_ref[...], preferred_element_type=jnp.float32)
                    + b2_ref[...])
    out_ref[...] = (jnp.dot(h, w3_ref[...], preferred_element_type=jnp.float32)
                    + b3_ref[...])


def kernel(x, edge_index, edge_attr, u, batch,
           m1_w1, m1_b1, m1_w2, m1_b2, m1_w3, m1_b3,
           m2_w1, m2_b1, m2_w2, m2_b2, m2_w3, m2_b3):
    row = edge_index[0].reshape(1, E)
    col = edge_index[1].reshape(1, E)
    pos = jnp.pad(x[:, :3], ((0, 0), (0, 13)))
    w1a = m1_w1[:C]
    w1b = jnp.pad(m1_w1[C:], ((0, 13), (0, 0)))

    full = lambda shape: pl.BlockSpec(shape, lambda i: (0,) * len(shape))
    grid_e = E // EB

    out_e = pl.pallas_call(
        _edge_mlp_kernel,
        grid=(grid_e,),
        in_specs=[
            pl.BlockSpec((1, EB), lambda i: (0, i), memory_space=pltpu.SMEM),
            pl.BlockSpec((1, EB), lambda i: (0, i), memory_space=pltpu.SMEM),
            full((N, C)), full((N, 16)),
            full((C, H)), full((16, H)), full((1, H)),
            full((H, H)), full((1, H)), full((H, L)), full((1, L)),
        ],
        out_specs=pl.BlockSpec((EB, L), lambda i: (i, 0)),
        out_shape=jax.ShapeDtypeStruct((E, L), jnp.float32),
        scratch_shapes=[pltpu.VMEM((EB, C), jnp.float32),
                        pltpu.VMEM((EB, 16), jnp.float32)],
    )(row, col, x, pos, w1a, w1b, m1_b1.reshape(1, H),
      m1_w2, m1_b2.reshape(1, H), m1_w3, m1_b3.reshape(1, L))

    s, mx, cnt = pl.pallas_call(
        _segment_kernel,
        grid=(grid_e,),
        in_specs=[
            pl.BlockSpec((1, EB), lambda i: (0, i), memory_space=pltpu.SMEM),
            pl.BlockSpec((EB, L), lambda i: (i, 0)),
        ],
        out_specs=[full((N, L)), full((N, L)), full((N, L))],
        out_shape=[jax.ShapeDtypeStruct((N, L), jnp.float32)] * 3,
    )(col, out_e)

    grid_n = N // NB
    out_n = pl.pallas_call(
        _node_mlp_kernel,
        grid=(grid_n,),
        in_specs=[
            pl.BlockSpec((NB, C), lambda i: (i, 0)),
            pl.BlockSpec((NB, L), lambda i: (i, 0)),
            pl.BlockSpec((NB, L), lambda i: (i, 0)),
            pl.BlockSpec((NB, L), lambda i: (i, 0)),
            full((C, H)), full((L, H)), full((L, H)), full((L, H)), full((1, H)),
            full((H, H)), full((1, H)), full((H, L)), full((1, L)),
        ],
        out_specs=pl.BlockSpec((NB, L), lambda i: (i, 0)),
        out_shape=jax.ShapeDtypeStruct((N, L), jnp.float32),
    )(x, s, mx, cnt,
      m2_w1[:C], m2_w1[C:C + L], m2_w1[C + L:C + 2 * L], m2_w1[C + 2 * L:],
      m2_b1.reshape(1, H), m2_w2, m2_b2.reshape(1, H),
      m2_w3, m2_b3.reshape(1, L))

    return jnp.concatenate([x[:, :3], out_n], axis=1)


# speedup vs baseline: 1.5214x; 1.5214x over previous
import functools
import jax
import jax.numpy as jnp
from jax import lax
from jax.experimental import pallas as pl
from jax.experimental.pallas import tpu as pltpu
from jax.experimental.pallas import tpu_sc as plsc

N = 10000
E = 320000
C = 128
H = 256
L = 128
EB = 1280
NB = 1000

GW = 32
EPW = E // GW
CH = 200
NCH = EPW // CH


def _gather_body(x_hbm, row_hbm, col_hbm, xg_hbm, ea_hbm,
                 idxr_v, idxc_v, rows_v, crows_v, ea_v, sem):
    wid = lax.axis_index("s") * 2 + lax.axis_index("c")
    base = wid * EPW

    def chunk(k, carry):
        off = base + k * CH
        pltpu.sync_copy(row_hbm.at[pl.ds(off, CH)], idxr_v)
        pltpu.sync_copy(col_hbm.at[pl.ds(off, CH)], idxc_v)
        pltpu.async_copy(x_hbm.at[idxr_v], rows_v, sem).wait()
        pltpu.async_copy(x_hbm.at[idxc_v], crows_v, sem).wait()

        def ea_row(r, c2):
            d = rows_v[r, pl.ds(0, 16)] - crows_v[r, pl.ds(0, 16)]
            d = jnp.where(d > 0.5, d - 1.0, d)
            d = jnp.where(-d > 0.5, d + 1.0, d)
            ea_v[r, pl.ds(0, 16)] = d
            return c2

        lax.fori_loop(0, CH, ea_row, 0)
        pltpu.sync_copy(rows_v, xg_hbm.at[pl.ds(off, CH)])
        pltpu.sync_copy(ea_v, ea_hbm.at[pl.ds(off, CH)])
        return carry

    lax.fori_loop(0, NCH, chunk, 0)


def _edge_mlp_kernel(xg_ref, ea_ref,
                     w1a_ref, w1b_ref, b1_ref, w2_ref, b2_ref, w3_ref, b3_ref,
                     out_ref):
    feat = xg_ref[...]
    ea = ea_ref[...]
    h = jnp.dot(feat, w1a_ref[...], preferred_element_type=jnp.float32)
    h = h + jnp.dot(ea, w1b_ref[...], preferred_element_type=jnp.float32)
    h = jax.nn.relu(h + b1_ref[...])
    h = jax.nn.relu(jnp.dot(h, w2_ref[...], preferred_element_type=jnp.float32)
                    + b2_ref[...])
    out_ref[...] = (jnp.dot(h, w3_ref[...], preferred_element_type=jnp.float32)
                    + b3_ref[...])


def _segment_kernel(col_ref, val_ref, s_ref, m_ref, c_ref):
    @pl.when(pl.program_id(0) == 0)
    def _():
        s_ref[...] = jnp.zeros_like(s_ref)
        m_ref[...] = jnp.full_like(m_ref, -3.0e38)
        c_ref[...] = jnp.zeros_like(c_ref)

    def body(i, _):
        c = col_ref[0, i]
        v = val_ref[pl.ds(i, 1), :]
        s_ref[pl.ds(c, 1), :] += v
        m_ref[pl.ds(c, 1), :] = jnp.maximum(m_ref[pl.ds(c, 1), :], v)
        c_ref[pl.ds(c, 1), :] += 1.0
        return 0

    lax.fori_loop(0, EB, body, 0)


def _node_mlp_kernel(x_ref, s_ref, m_ref, c_ref,
                     wa_ref, wb_ref, wc_ref, wd_ref, b1_ref,
                     w2_ref, b2_ref, w3_ref, b3_ref, out_ref):
    cnt = c_ref[...]
    s = s_ref[...]
    mean = s / jnp.maximum(cnt, 1.0)
    mx = jnp.where(cnt > 0, m_ref[...], 0.0)
    h = jnp.dot(x_ref[...], wa_ref[...], preferred_element_type=jnp.float32)
    h = h + jnp.dot(mean, wb_ref[...], preferred_element_type=jnp.float32)
    h = h + jnp.dot(mx, wc_ref[...], preferred_element_type=jnp.float32)
    h = h + jnp.dot(s, wd_ref[...], preferred_element_type=jnp.float32)
    h = jax.nn.relu(h + b1_ref[...])
    h = jax.nn.relu(jnp.dot(h, w2_ref[...], preferred_element_type=jnp.float32)
                    + b2_ref[...])
    out_ref[...] = (jnp.dot(h, w3_ref[...], preferred_element_type=jnp.float32)
                    + b3_ref[...])


def kernel(x, edge_index, edge_attr, u, batch,
           m1_w1, m1_b1, m1_w2, m1_b2, m1_w3, m1_b3,
           m2_w1, m2_b1, m2_w2, m2_b2, m2_w3, m2_b3):
    row = edge_index[0]
    col = edge_index[1]
    w1a = m1_w1[:C]
    w1b = jnp.pad(m1_w1[C:], ((0, 13), (0, 0)))

    gather = functools.partial(
        pl.kernel,
        out_type=[jax.ShapeDtypeStruct((E, C), jnp.float32),
                  jax.ShapeDtypeStruct((E, 16), jnp.float32)],
        mesh=plsc.VectorSubcoreMesh(core_axis_name="c", subcore_axis_name="s"),
        scratch_types=[pltpu.VMEM((CH,), jnp.int32),
                       pltpu.VMEM((CH,), jnp.int32),
                       pltpu.VMEM((CH, C), jnp.float32),
                       pltpu.VMEM((CH, C), jnp.float32),
                       pltpu.VMEM((CH, 16), jnp.float32),
                       pltpu.SemaphoreType.DMA],
    )(_gather_body)
    xg, ea = gather(x, row, col)

    full = lambda shape: pl.BlockSpec(shape, lambda i: (0,) * len(shape))
    grid_e = E // EB

    out_e = pl.pallas_call(
        _edge_mlp_kernel,
        grid=(grid_e,),
        in_specs=[
            pl.BlockSpec((EB, C), lambda i: (i, 0)),
            pl.BlockSpec((EB, 16), lambda i: (i, 0)),
            full((C, H)), full((16, H)), full((1, H)),
            full((H, H)), full((1, H)), full((H, L)), full((1, L)),
        ],
        out_specs=pl.BlockSpec((EB, L), lambda i: (i, 0)),
        out_shape=jax.ShapeDtypeStruct((E, L), jnp.float32),
    )(xg, ea, w1a, w1b, m1_b1.reshape(1, H),
      m1_w2, m1_b2.reshape(1, H), m1_w3, m1_b3.reshape(1, L))

    s, mx, cnt = pl.pallas_call(
        _segment_kernel,
        grid=(grid_e,),
        in_specs=[
            pl.BlockSpec((1, EB), lambda i: (0, i), memory_space=pltpu.SMEM),
            pl.BlockSpec((EB, L), lambda i: (i, 0)),
        ],
        out_specs=[full((N, L)), full((N, L)), full((N, L))],
        out_shape=[jax.ShapeDtypeStruct((N, L), jnp.float32)] * 3,
    )(col.reshape(1, E), out_e)

    grid_n = N // NB
    out_n = pl.pallas_call(
        _node_mlp_kernel,
        grid=(grid_n,),
        in_specs=[
            pl.BlockSpec((NB, C), lambda i: (i, 0)),
            pl.BlockSpec((NB, L), lambda i: (i, 0)),
            pl.BlockSpec((NB, L), lambda i: (i, 0)),
            pl.BlockSpec((NB, L), lambda i: (i, 0)),
            full((C, H)), full((L, H)), full((L, H)), full((L, H)), full((1, H)),
            full((H, H)), full((1, H)), full((H, L)), full((1, L)),
        ],
        out_specs=pl.BlockSpec((NB, L), lambda i: (i, 0)),
        out_shape=jax.ShapeDtypeStruct((N, L), jnp.float32),
    )(x, s, mx, cnt,
      m2_w1[:C], m2_w1[C:C + L], m2_w1[C + L:C + 2 * L], m2_w1[C + 2 * L:],
      m2_b1.reshape(1, H), m2_w2, m2_b2.reshape(1, H),
      m2_w3, m2_b3.reshape(1, L))

    return jnp.concatenate([x[:, :3], out_n], axis=1)


# speedup vs baseline: 2.6725x; 1.7567x over previous
import functools
import jax
import jax.numpy as jnp
from jax import lax
from jax.experimental import pallas as pl
from jax.experimental.pallas import tpu as pltpu
from jax.experimental.pallas import tpu_sc as plsc

N = 10000
E = 320000
C = 128
H = 256
L = 128
EB = 1280
NB = 1000

GW = 32
EPW = E // GW
CH = 200
NCH = EPW // CH


def _gather_body(x_hbm, row_hbm, col_hbm, xg_hbm, ea_hbm,
                 idxr_v, idxc_v, rows_v, crows_v, ea_v, sem):
    wid = lax.axis_index("s") * 2 + lax.axis_index("c")
    base = wid * EPW

    def chunk(k, carry):
        off = base + k * CH
        pltpu.sync_copy(row_hbm.at[pl.ds(off, CH)], idxr_v)
        pltpu.sync_copy(col_hbm.at[pl.ds(off, CH)], idxc_v)
        pltpu.async_copy(x_hbm.at[idxr_v], rows_v, sem).wait()
        pltpu.async_copy(x_hbm.at[idxc_v], crows_v, sem).wait()

        def ea_row(r, c2):
            d = rows_v[r, pl.ds(0, 16)] - crows_v[r, pl.ds(0, 16)]
            d = jnp.where(d > 0.5, d - 1.0, d)
            d = jnp.where(-d > 0.5, d + 1.0, d)
            ea_v[r, pl.ds(0, 16)] = d
            return c2

        lax.fori_loop(0, CH, ea_row, 0)
        pltpu.sync_copy(rows_v, xg_hbm.at[pl.ds(off, CH)])
        pltpu.sync_copy(ea_v, ea_hbm.at[pl.ds(off, CH)])
        return carry

    lax.fori_loop(0, NCH, chunk, 0)


def _edge_mlp_kernel(xg_ref, ea_ref,
                     w1a_ref, w1b_ref, b1_ref, w2_ref, b2_ref, w3_ref, b3_ref,
                     out_ref):
    feat = xg_ref[...]
    ea = ea_ref[...]
    h = jnp.dot(feat, w1a_ref[...], preferred_element_type=jnp.float32)
    h = h + jnp.dot(ea, w1b_ref[...], preferred_element_type=jnp.float32)
    h = jax.nn.relu(h + b1_ref[...])
    h = jax.nn.relu(jnp.dot(h, w2_ref[...], preferred_element_type=jnp.float32)
                    + b2_ref[...])
    out_ref[...] = (jnp.dot(h, w3_ref[...], preferred_element_type=jnp.float32)
                    + b3_ref[...])


def _segment_kernel(col_ref, val_ref,
                    s0, s1, s2, s3, m0, m1, m2, m3, c0, c1):
    s_refs = (s0, s1, s2, s3)
    m_refs = (m0, m1, m2, m3)
    c_refs = (c0, c1)

    @pl.when(pl.program_id(0) == 0)
    def _():
        for r in s_refs + c_refs:
            r[...] = jnp.zeros_like(r)
        for r in m_refs:
            r[...] = jnp.full_like(r, -3.0e38)

    def body(i, _):
        for j in range(4):
            e = i * 4 + j
            c = col_ref[0, e]
            v = val_ref[pl.ds(e, 1), :]
            s_refs[j][pl.ds(c, 1), :] += v
            m_refs[j][pl.ds(c, 1), :] = jnp.maximum(m_refs[j][pl.ds(c, 1), :], v)
            c_refs[j % 2][pl.ds(c, 1), :] += 1.0
        return 0

    lax.fori_loop(0, EB // 4, body, 0)


def _node_mlp_kernel(x_ref, s0, s1, s2, s3, m0, m1, m2, m3, c0, c1,
                     wa_ref, wb_ref, wc_ref, wd_ref, b1_ref,
                     w2_ref, b2_ref, w3_ref, b3_ref, out_ref):
    cnt = c0[...] + c1[...]
    s = (s0[...] + s1[...]) + (s2[...] + s3[...])
    mraw = jnp.maximum(jnp.maximum(m0[...], m1[...]),
                       jnp.maximum(m2[...], m3[...]))
    mean = s / jnp.maximum(cnt, 1.0)
    mx = jnp.where(cnt > 0, mraw, 0.0)
    h = jnp.dot(x_ref[...], wa_ref[...], preferred_element_type=jnp.float32)
    h = h + jnp.dot(mean, wb_ref[...], preferred_element_type=jnp.float32)
    h = h + jnp.dot(mx, wc_ref[...], preferred_element_type=jnp.float32)
    h = h + jnp.dot(s, wd_ref[...], preferred_element_type=jnp.float32)
    h = jax.nn.relu(h + b1_ref[...])
    h = jax.nn.relu(jnp.dot(h, w2_ref[...], preferred_element_type=jnp.float32)
                    + b2_ref[...])
    out_ref[...] = (jnp.dot(h, w3_ref[...], preferred_element_type=jnp.float32)
                    + b3_ref[...])


def kernel(x, edge_index, edge_attr, u, batch,
           m1_w1, m1_b1, m1_w2, m1_b2, m1_w3, m1_b3,
           m2_w1, m2_b1, m2_w2, m2_b2, m2_w3, m2_b3):
    row = edge_index[0]
    col = edge_index[1]
    w1a = m1_w1[:C]
    w1b = jnp.pad(m1_w1[C:], ((0, 13), (0, 0)))

    gather = functools.partial(
        pl.kernel,
        out_type=[jax.ShapeDtypeStruct((E, C), jnp.float32),
                  jax.ShapeDtypeStruct((E, 16), jnp.float32)],
        mesh=plsc.VectorSubcoreMesh(core_axis_name="c", subcore_axis_name="s"),
        scratch_types=[pltpu.VMEM((CH,), jnp.int32),
                       pltpu.VMEM((CH,), jnp.int32),
                       pltpu.VMEM((CH, C), jnp.float32),
                       pltpu.VMEM((CH, C), jnp.float32),
                       pltpu.VMEM((CH, 16), jnp.float32),
                       pltpu.SemaphoreType.DMA],
    )(_gather_body)
    xg, ea = gather(x, row, col)

    full = lambda shape: pl.BlockSpec(shape, lambda i: (0,) * len(shape))
    grid_e = E // EB

    out_e = pl.pallas_call(
        _edge_mlp_kernel,
        grid=(grid_e,),
        in_specs=[
            pl.BlockSpec((EB, C), lambda i: (i, 0)),
            pl.BlockSpec((EB, 16), lambda i: (i, 0)),
            full((C, H)), full((16, H)), full((1, H)),
            full((H, H)), full((1, H)), full((H, L)), full((1, L)),
        ],
        out_specs=pl.BlockSpec((EB, L), lambda i: (i, 0)),
        out_shape=jax.ShapeDtypeStruct((E, L), jnp.float32),
    )(xg, ea, w1a, w1b, m1_b1.reshape(1, H),
      m1_w2, m1_b2.reshape(1, H), m1_w3, m1_b3.reshape(1, L))

    segs = pl.pallas_call(
        _segment_kernel,
        grid=(grid_e,),
        in_specs=[
            pl.BlockSpec((1, EB), lambda i: (0, i), memory_space=pltpu.SMEM),
            pl.BlockSpec((EB, L), lambda i: (i, 0)),
        ],
        out_specs=[full((N, L))] * 10,
        out_shape=[jax.ShapeDtypeStruct((N, L), jnp.float32)] * 10,
    )(col.reshape(1, E), out_e)

    grid_n = N // NB
    out_n = pl.pallas_call(
        _node_mlp_kernel,
        grid=(grid_n,),
        in_specs=[
            pl.BlockSpec((NB, C), lambda i: (i, 0)),
        ] + [pl.BlockSpec((NB, L), lambda i: (i, 0))] * 10 + [
            full((C, H)), full((L, H)), full((L, H)), full((L, H)), full((1, H)),
            full((H, H)), full((1, H)), full((H, L)), full((1, L)),
        ],
        out_specs=pl.BlockSpec((NB, L), lambda i: (i, 0)),
        out_shape=jax.ShapeDtypeStruct((N, L), jnp.float32),
    )(x, *segs,
      m2_w1[:C], m2_w1[C:C + L], m2_w1[C + L:C + 2 * L], m2_w1[C + 2 * L:],
      m2_b1.reshape(1, H), m2_w2, m2_b2.reshape(1, H),
      m2_w3, m2_b3.reshape(1, L))

    return jnp.concatenate([x[:, :3], out_n], axis=1)
